# baseline (speedup 1.0000x reference)
import numpy as np
import jax
import jax.numpy as jnp
from jax.experimental import pallas as pl
from jax.experimental.pallas import tpu as pltpu

_ALPHA = 4


def _pack_body(sel_ref, pos_ref, src_ref, fast_ref, slow_ref):
    fast_ref[...] = src_ref[...]
    t = pl.program_id(1)

    @pl.when(sel_ref[t] != 0)
    def _():
        slow_ref[...] = src_ref[...]

    del pos_ref


def kernel(frames):
    temporal_axis = 1 if frames.ndim == 4 else 2
    T = frames.shape[temporal_axis]
    S = T // _ALPHA
    idx = tuple(int(v) for v in np.linspace(0.0, T - 1, S))

    if frames.ndim == 4:
        C, _, H, W = frames.shape
        lead = C
    else:
        B, C, _, H, W = frames.shape
        lead = B * C

    sel = np.zeros((T,), dtype=np.int32)
    pos = np.zeros((T,), dtype=np.int32)
    p = -1
    sel[list(idx)] = 1
    for t in range(T):
        if sel[t]:
            p += 1
        pos[t] = max(p, 0)

    hw = H * W
    lanes = 128
    rows = hw // lanes
    x = frames.reshape(lead, T, rows, lanes)

    fast, slow = pl.pallas_call(
        _pack_body,
        grid_spec=pltpu.PrefetchScalarGridSpec(
            num_scalar_prefetch=2,
            grid=(lead, T),
            in_specs=[
                pl.BlockSpec((1, 1, rows, lanes),
                             lambda i, t, sel_ref, pos_ref: (i, t, 0, 0)),
            ],
            out_specs=[
                pl.BlockSpec((1, 1, rows, lanes),
                             lambda i, t, sel_ref, pos_ref: (i, t, 0, 0)),
                pl.BlockSpec((1, 1, rows, lanes),
                             lambda i, t, sel_ref, pos_ref: (i, pos_ref[t], 0, 0)),
            ],
        ),
        out_shape=[
            jax.ShapeDtypeStruct((lead, T, rows, lanes), frames.dtype),
            jax.ShapeDtypeStruct((lead, S, rows, lanes), frames.dtype),
        ],
    )(jnp.asarray(sel), jnp.asarray(pos), x)

    if frames.ndim == 4:
        slow = slow.reshape(C, S, H, W)
        fast = fast.reshape(C, T, H, W)
    else:
        slow = slow.reshape(B, C, S, H, W)
        fast = fast.reshape(B, C, T, H, W)
    return (slow, fast)


# speedup vs baseline: 1.4661x; 1.4661x over previous
import numpy as np
import jax
import jax.numpy as jnp
from jax.experimental import pallas as pl

_ALPHA = 4
_LANES = 128
_ROW_CHUNK = 56


def _make_body(idx):
    def _pack_body(src_ref, fast_ref, slow_ref):
        fast_ref[...] = src_ref[...]
        for k, t in enumerate(idx):
            slow_ref[:, k] = src_ref[:, t]
    return _pack_body


def kernel(frames):
    temporal_axis = 1 if frames.ndim == 4 else 2
    T = frames.shape[temporal_axis]
    S = T // _ALPHA
    idx = tuple(int(v) for v in np.linspace(0.0, T - 1, S))

    if frames.ndim == 4:
        C, _, H, W = frames.shape
        lead = C
    else:
        B, C, _, H, W = frames.shape
        lead = B * C

    hw = H * W
    rows = hw // _LANES
    rc = _ROW_CHUNK if rows % _ROW_CHUNK == 0 else rows
    n_chunks = rows // rc
    x = frames.reshape(lead, T, rows, _LANES)

    fast, slow = pl.pallas_call(
        _make_body(idx),
        grid=(lead, n_chunks),
        in_specs=[
            pl.BlockSpec((1, T, rc, _LANES), lambda i, j: (i, 0, j, 0)),
        ],
        out_specs=[
            pl.BlockSpec((1, T, rc, _LANES), lambda i, j: (i, 0, j, 0)),
            pl.BlockSpec((1, S, rc, _LANES), lambda i, j: (i, 0, j, 0)),
        ],
        out_shape=[
            jax.ShapeDtypeStruct((lead, T, rows, _LANES), frames.dtype),
            jax.ShapeDtypeStruct((lead, S, rows, _LANES), frames.dtype),
        ],
    )(x)

    if frames.ndim == 4:
        slow = slow.reshape(C, S, H, W)
        fast = fast.reshape(C, T, H, W)
    else:
        slow = slow.reshape(B, C, S, H, W)
        fast = fast.reshape(B, C, T, H, W)
    return (slow, fast)


# speedup vs baseline: 1.6326x; 1.1136x over previous
import numpy as np
import jax
import jax.numpy as jnp
from jax.experimental import pallas as pl

_ALPHA = 4
_LANES = 128
_ROW_CHUNK = 392


def _make_body(idx):
    def _pack_body(src_ref, fast_ref, slow_ref):
        fast_ref[...] = src_ref[...]
        for k, t in enumerate(idx):
            slow_ref[:, k] = src_ref[:, t]
    return _pack_body


def kernel(frames):
    temporal_axis = 1 if frames.ndim == 4 else 2
    T = frames.shape[temporal_axis]
    S = T // _ALPHA
    idx = tuple(int(v) for v in np.linspace(0.0, T - 1, S))

    if frames.ndim == 4:
        C, _, H, W = frames.shape
        lead = C
    else:
        B, C, _, H, W = frames.shape
        lead = B * C

    hw = H * W
    rows = hw // _LANES
    rc = _ROW_CHUNK if rows % _ROW_CHUNK == 0 else rows
    n_chunks = rows // rc
    x = frames.reshape(lead, T, rows, _LANES)

    fast, slow = pl.pallas_call(
        _make_body(idx),
        grid=(lead, n_chunks),
        in_specs=[
            pl.BlockSpec((1, T, rc, _LANES), lambda i, j: (i, 0, j, 0)),
        ],
        out_specs=[
            pl.BlockSpec((1, T, rc, _LANES), lambda i, j: (i, 0, j, 0)),
            pl.BlockSpec((1, S, rc, _LANES), lambda i, j: (i, 0, j, 0)),
        ],
        out_shape=[
            jax.ShapeDtypeStruct((lead, T, rows, _LANES), frames.dtype),
            jax.ShapeDtypeStruct((lead, S, rows, _LANES), frames.dtype),
        ],
    )(x)

    if frames.ndim == 4:
        slow = slow.reshape(C, S, H, W)
        fast = fast.reshape(C, T, H, W)
    else:
        slow = slow.reshape(B, C, S, H, W)
        fast = fast.reshape(B, C, T, H, W)
    return (slow, fast)


# speedup vs baseline: 2.2446x; 1.3748x over previous
import numpy as np
import jax
import jax.numpy as jnp
from jax.experimental import pallas as pl

_ALPHA = 4
_LANES = 128


def _gather_body(*refs):
    srcs, out = refs[:-1], refs[-1]
    for k, s in enumerate(srcs):
        out[:, k] = s[:, 0]


def kernel(frames):
    temporal_axis = 1 if frames.ndim == 4 else 2
    T = frames.shape[temporal_axis]
    S = T // _ALPHA
    idx = tuple(int(v) for v in np.linspace(0.0, T - 1, S))

    if frames.ndim == 4:
        C, _, H, W = frames.shape
        lead = C
    else:
        B, C, _, H, W = frames.shape
        lead = B * C

    hw = H * W
    rows = hw // _LANES
    x = frames.reshape(lead, T, rows, _LANES)

    def _spec(t):
        return pl.BlockSpec((1, 1, rows, _LANES), lambda i, _t=t: (i, _t, 0, 0))

    slow = pl.pallas_call(
        _gather_body,
        grid=(lead,),
        in_specs=[_spec(t) for t in idx],
        out_specs=pl.BlockSpec((1, S, rows, _LANES), lambda i: (i, 0, 0, 0)),
        out_shape=jax.ShapeDtypeStruct((lead, S, rows, _LANES), frames.dtype),
    )(*([x] * S))

    if frames.ndim == 4:
        slow = slow.reshape(C, S, H, W)
    else:
        slow = slow.reshape(B, C, S, H, W)
    return (slow, frames)
